# baseline (speedup 1.0000x reference)
import functools

import jax
import jax.numpy as jnp
from jax import lax
from jax.experimental import pallas as pl
from jax.experimental.pallas import tpu as pltpu
from jax.experimental.pallas import tpu_sc as plsc

_NC, _NS, _L = 2, 16, 16
_NW = _NC * _NS


def _sc_positions(entity_list):
    B, T = entity_list.shape
    rows_w = B // _NW
    G = rows_w // _L
    n = rows_w * T
    mesh = plsc.VectorSubcoreMesh(core_axis_name="c", subcore_axis_name="s")

    @functools.partial(
        pl.kernel,
        out_type=jax.ShapeDtypeStruct((B * T,), jnp.int32),
        mesh=mesh,
        scratch_types=[pltpu.VMEM((n,), jnp.int32)],
        compiler_params=pltpu.CompilerParams(needs_layout_passes=False),
    )
    def pos_kernel(el_hbm, pos_hbm, buf):
        wid = lax.axis_index("s") * _NC + lax.axis_index("c")
        base = wid * n
        pltpu.sync_copy(el_hbm.at[pl.ds(base, n)], buf)
        lane = lax.iota(jnp.int32, 16)

        def group(g, _):
            row0 = (g * _L + lane) * T

            def step(t, carry):
                idx = row0 + t
                x = plsc.load_gather(buf, [idx])
                mi = (x != 0).astype(jnp.int32)
                carry = carry + mi
                plsc.store_scatter(buf, [idx], carry * mi)
                return carry

            lax.fori_loop(0, T, step, jnp.zeros((16,), jnp.int32))
            return 0

        lax.fori_loop(0, G, group, 0)
        pltpu.sync_copy(buf, pos_hbm.at[pl.ds(base, n)])

    return pos_kernel(entity_list.reshape(B * T)).reshape(B, T)


def _tc_body(pos_ref, emb_ref, w2_ref, out_ref):
    R, T = pos_ref.shape
    _, V, D2 = w2_ref.shape
    H = T // 2
    pos = pos_ref[...].astype(jnp.bfloat16)
    s_r = lax.broadcasted_iota(jnp.int32, (T, H), 0)
    s_c = lax.broadcasted_iota(jnp.int32, (T, H), 1)
    se = (s_r == 2 * s_c).astype(jnp.bfloat16)
    so = (s_r == 2 * s_c + 1).astype(jnp.bfloat16)
    pos_e = jnp.dot(pos, se, preferred_element_type=jnp.float32).astype(jnp.bfloat16)
    pos_o = jnp.dot(pos, so, preferred_element_type=jnp.float32).astype(jnp.bfloat16)
    vi = lax.broadcasted_iota(jnp.int32, (1, 1, V), 2).astype(jnp.bfloat16)
    one = jnp.bfloat16(1)
    zero = jnp.bfloat16(0)
    oh_e = jnp.where(pos_e[:, :, None] == vi, one, zero).reshape(R * H, V)
    oh_o = jnp.where(pos_o[:, :, None] == vi, one, zero).reshape(R * H, V)
    pe = (jnp.dot(oh_e, w2_ref[0], preferred_element_type=jnp.float32)
          + jnp.dot(oh_o, w2_ref[1], preferred_element_type=jnp.float32))
    out_ref[...] = emb_ref[...] + pe.reshape(R, H, D2)


def kernel(entity_embeds, entity_list, W):
    B, T, D = entity_embeds.shape
    V = W.shape[0]
    H = T // 2
    R = min(64, B)
    pos = _sc_positions(entity_list)
    wb = W.astype(jnp.bfloat16)
    z = jnp.zeros_like(wb)
    w2 = jnp.stack([jnp.concatenate([wb, z], axis=1),
                    jnp.concatenate([z, wb], axis=1)])
    emb2 = entity_embeds.reshape(B, H, 2 * D)
    out = pl.pallas_call(
        _tc_body,
        grid=(B // R,),
        in_specs=[
            pl.BlockSpec((R, T), lambda i: (i, 0)),
            pl.BlockSpec((R, H, 2 * D), lambda i: (i, 0, 0)),
            pl.BlockSpec((2, V, 2 * D), lambda i: (0, 0, 0)),
        ],
        out_specs=pl.BlockSpec((R, H, 2 * D), lambda i: (i, 0, 0)),
        out_shape=jax.ShapeDtypeStruct((B, H, 2 * D), jnp.float32),
    )(pos, emb2, w2)
    return out.reshape(B, T, D)


# speedup vs baseline: 1.0463x; 1.0463x over previous
import functools

import jax
import jax.numpy as jnp
from jax import lax
from jax.experimental import pallas as pl
from jax.experimental.pallas import tpu as pltpu
from jax.experimental.pallas import tpu_sc as plsc

_NC, _NS, _L = 2, 16, 16
_NW = _NC * _NS


def _sc_positions(entity_list):
    B, T = entity_list.shape
    rows_w = B // _NW
    G = rows_w // _L
    n = rows_w * T
    mesh = plsc.VectorSubcoreMesh(core_axis_name="c", subcore_axis_name="s")

    @functools.partial(
        pl.kernel,
        out_type=jax.ShapeDtypeStruct((B * T,), jnp.int32),
        mesh=mesh,
        scratch_types=[pltpu.VMEM((n,), jnp.int32)],
        compiler_params=pltpu.CompilerParams(needs_layout_passes=False),
    )
    def pos_kernel(el_hbm, pos_hbm, buf):
        wid = lax.axis_index("s") * _NC + lax.axis_index("c")
        base = wid * n
        pltpu.sync_copy(el_hbm.at[pl.ds(base, n)], buf)
        lane = lax.iota(jnp.int32, 16)

        def group(g, _):
            row0 = (g * _L + lane) * T

            def step(t, carry):
                idx = row0 + t
                x = plsc.load_gather(buf, [idx])
                mi = (x != 0).astype(jnp.int32)
                carry = carry + mi
                plsc.store_scatter(buf, [idx], carry * mi)
                return carry

            lax.fori_loop(0, T, step, jnp.zeros((16,), jnp.int32))
            return 0

        lax.fori_loop(0, G, group, 0)
        pltpu.sync_copy(buf, pos_hbm.at[pl.ds(base, n)])

    return pos_kernel(entity_list.reshape(B * T)).reshape(B, T)


def _tc_body(pos_ref, emb_ref, w2_ref, out_ref):
    R, T = pos_ref.shape
    _, V, D2 = w2_ref.shape
    H = T // 2
    pos = pos_ref[...].astype(jnp.bfloat16)
    s_r = lax.broadcasted_iota(jnp.int32, (T, H), 0)
    s_c = lax.broadcasted_iota(jnp.int32, (T, H), 1)
    se = (s_r == 2 * s_c).astype(jnp.bfloat16)
    so = (s_r == 2 * s_c + 1).astype(jnp.bfloat16)
    pos_e = jnp.dot(pos, se, preferred_element_type=jnp.float32).astype(jnp.bfloat16)
    pos_o = jnp.dot(pos, so, preferred_element_type=jnp.float32).astype(jnp.bfloat16)
    vi = lax.broadcasted_iota(jnp.int32, (1, 1, V), 2).astype(jnp.bfloat16)
    one = jnp.bfloat16(1)
    zero = jnp.bfloat16(0)
    oh_e = jnp.where(pos_e[:, :, None] == vi, one, zero).reshape(R * H, V)
    oh_o = jnp.where(pos_o[:, :, None] == vi, one, zero).reshape(R * H, V)
    pe = (jnp.dot(oh_e, w2_ref[0], preferred_element_type=jnp.float32)
          + jnp.dot(oh_o, w2_ref[1], preferred_element_type=jnp.float32))
    out_ref[...] = emb_ref[...] + pe.reshape(R, H, D2)


def kernel(entity_embeds, entity_list, W):
    B, T, D = entity_embeds.shape
    V = W.shape[0]
    H = T // 2
    R = min(128, B)
    pos = _sc_positions(entity_list)
    wb = W.astype(jnp.bfloat16)
    z = jnp.zeros_like(wb)
    w2 = jnp.stack([jnp.concatenate([wb, z], axis=1),
                    jnp.concatenate([z, wb], axis=1)])
    emb2 = entity_embeds.reshape(B, H, 2 * D)
    out = pl.pallas_call(
        _tc_body,
        grid=(B // R,),
        in_specs=[
            pl.BlockSpec((R, T), lambda i: (i, 0)),
            pl.BlockSpec((R, H, 2 * D), lambda i: (i, 0, 0)),
            pl.BlockSpec((2, V, 2 * D), lambda i: (0, 0, 0)),
        ],
        out_specs=pl.BlockSpec((R, H, 2 * D), lambda i: (i, 0, 0)),
        out_shape=jax.ShapeDtypeStruct((B, H, 2 * D), jnp.float32),
    )(pos, emb2, w2)
    return out.reshape(B, T, D)


# speedup vs baseline: 1.0738x; 1.0263x over previous
import functools

import jax
import jax.numpy as jnp
from jax import lax
from jax.experimental import pallas as pl
from jax.experimental.pallas import tpu as pltpu
from jax.experimental.pallas import tpu_sc as plsc

_NC, _NS, _L = 2, 16, 16
_NW = _NC * _NS


def _sc_positions(entity_list):
    B, T = entity_list.shape
    rows_w = B // _NW
    G = rows_w // _L
    n = rows_w * T
    mesh = plsc.VectorSubcoreMesh(core_axis_name="c", subcore_axis_name="s")

    @functools.partial(
        pl.kernel,
        out_type=jax.ShapeDtypeStruct((B * T,), jnp.int32),
        mesh=mesh,
        scratch_types=[pltpu.VMEM((n,), jnp.int32)],
        compiler_params=pltpu.CompilerParams(needs_layout_passes=False),
    )
    def pos_kernel(el_hbm, pos_hbm, buf):
        wid = lax.axis_index("s") * _NC + lax.axis_index("c")
        base = wid * n
        pltpu.sync_copy(el_hbm.at[pl.ds(base, n)], buf)
        lane = lax.iota(jnp.int32, 16)

        def group(g, _):
            row0 = (g * _L + lane) * T

            def step(t, carry):
                idx = row0 + t
                x = plsc.load_gather(buf, [idx])
                mi = (x != 0).astype(jnp.int32)
                carry = carry + mi
                plsc.store_scatter(buf, [idx], carry * mi)
                return carry

            lax.fori_loop(0, T, step, jnp.zeros((16,), jnp.int32))
            return 0

        lax.fori_loop(0, G, group, 0)
        pltpu.sync_copy(buf, pos_hbm.at[pl.ds(base, n)])

    return pos_kernel(entity_list.reshape(B * T)).reshape(B, T)


def _tc_body(pos_ref, emb_ref, w2_ref, out_ref):
    R, T = pos_ref.shape
    _, V, D2 = w2_ref.shape
    H = T // 2
    pos = pos_ref[...].astype(jnp.bfloat16)
    s_r = lax.broadcasted_iota(jnp.int32, (T, H), 0)
    s_c = lax.broadcasted_iota(jnp.int32, (T, H), 1)
    se = (s_r == 2 * s_c).astype(jnp.bfloat16)
    so = (s_r == 2 * s_c + 1).astype(jnp.bfloat16)
    pos_e = jnp.dot(pos, se, preferred_element_type=jnp.float32)
    pos_o = jnp.dot(pos, so, preferred_element_type=jnp.float32)
    vi = lax.broadcasted_iota(jnp.int32, (1, 1, V), 2).astype(jnp.float32)
    one = jnp.float32(1)
    zero = jnp.float32(0)
    oh_e = jnp.where(pos_e[:, :, None] == vi, one, zero).reshape(R * H, V).astype(jnp.bfloat16)
    oh_o = jnp.where(pos_o[:, :, None] == vi, one, zero).reshape(R * H, V).astype(jnp.bfloat16)
    pe = (jnp.dot(oh_e, w2_ref[0], preferred_element_type=jnp.float32)
          + jnp.dot(oh_o, w2_ref[1], preferred_element_type=jnp.float32))
    out_ref[...] = emb_ref[...] + pe.reshape(R, H, D2)


def kernel(entity_embeds, entity_list, W):
    B, T, D = entity_embeds.shape
    V = W.shape[0]
    H = T // 2
    R = min(128, B)
    pos = _sc_positions(entity_list)
    wb = W.astype(jnp.bfloat16)
    z = jnp.zeros_like(wb)
    w2 = jnp.stack([jnp.concatenate([wb, z], axis=1),
                    jnp.concatenate([z, wb], axis=1)])
    emb2 = entity_embeds.reshape(B, H, 2 * D)
    out = pl.pallas_call(
        _tc_body,
        grid=(B // R,),
        in_specs=[
            pl.BlockSpec((R, T), lambda i: (i, 0)),
            pl.BlockSpec((R, H, 2 * D), lambda i: (i, 0, 0)),
            pl.BlockSpec((2, V, 2 * D), lambda i: (0, 0, 0)),
        ],
        out_specs=pl.BlockSpec((R, H, 2 * D), lambda i: (i, 0, 0)),
        out_shape=jax.ShapeDtypeStruct((B, H, 2 * D), jnp.float32),
    )(pos, emb2, w2)
    return out.reshape(B, T, D)
